# baseline (speedup 1.0000x reference)
import jax
import jax.numpy as jnp
from jax.experimental import pallas as pl
from jax.experimental.pallas import tpu as pltpu

_NCLS = 512
_BB = 128


def _glvq_body(x_ref, p_ref, o_ref):
    x = x_ref[:]
    pa = p_ref[:_NCLS, :]
    pb = p_ref[_NCLS:, :]
    xx = jnp.sum(x * x, axis=1, keepdims=True)
    dn = (((1,), (1,)), ((), ()))
    xa = jax.lax.dot_general(x, pa, dn, preferred_element_type=jnp.float32)
    xb = jax.lax.dot_general(x, pb, dn, preferred_element_type=jnp.float32)
    da = xx - 2.0 * xa + jnp.sum(pa * pa, axis=1)[None, :]
    db = xx - 2.0 * xb + jnp.sum(pb * pb, axis=1)[None, :]
    o_ref[:] = jnp.sqrt(jnp.maximum(jnp.minimum(da, db), 0.0))


def kernel(x, protos):
    batch, d = x.shape
    nb = batch // _BB
    return pl.pallas_call(
        _glvq_body,
        grid=(nb,),
        in_specs=[
            pl.BlockSpec((_BB, d), lambda i: (i, 0)),
            pl.BlockSpec(protos.shape, lambda i: (0, 0)),
        ],
        out_specs=pl.BlockSpec((_BB, _NCLS), lambda i: (i, 0)),
        out_shape=jax.ShapeDtypeStruct((batch, _NCLS), jnp.float32),
    )(x, protos)


# speedup vs baseline: 1.3512x; 1.3512x over previous
import jax
import jax.numpy as jnp
from jax.experimental import pallas as pl
from jax.experimental.pallas import tpu as pltpu

_NCLS = 512


def _glvq_body(x_ref, p_ref, o_ref):
    x = x_ref[:]
    pa = p_ref[:_NCLS, :]
    pb = p_ref[_NCLS:, :]
    xx = jnp.sum(x * x, axis=1, keepdims=True)
    dn = (((1,), (1,)), ((), ()))
    xa = jax.lax.dot_general(x, pa, dn, preferred_element_type=jnp.float32)
    xb = jax.lax.dot_general(x, pb, dn, preferred_element_type=jnp.float32)
    da = xx - 2.0 * xa + jnp.sum(pa * pa, axis=1)[None, :]
    db = xx - 2.0 * xb + jnp.sum(pb * pb, axis=1)[None, :]
    o_ref[:] = jnp.sqrt(jnp.maximum(jnp.minimum(da, db), 0.0))


def kernel(x, protos):
    batch = x.shape[0]
    return pl.pallas_call(
        _glvq_body,
        out_shape=jax.ShapeDtypeStruct((batch, _NCLS), jnp.float32),
    )(x, protos)


# speedup vs baseline: 1.4118x; 1.0449x over previous
import jax
import jax.numpy as jnp
from jax.experimental import pallas as pl
from jax.experimental.pallas import tpu as pltpu

_NCLS = 512


def _glvq_body(x_ref, p_ref, o_ref):
    x = x_ref[:]
    pa = p_ref[:_NCLS, :]
    pb = p_ref[_NCLS:, :]
    xm2 = x * -2.0
    xx = jnp.sum(x * x, axis=1, keepdims=True)
    dn = (((1,), (1,)), ((), ()))
    na = jax.lax.dot_general(xm2, pa, dn, preferred_element_type=jnp.float32)
    nb = jax.lax.dot_general(xm2, pb, dn, preferred_element_type=jnp.float32)
    na = na + jnp.sum(pa * pa, axis=1)[None, :]
    nb = nb + jnp.sum(pb * pb, axis=1)[None, :]
    m = jnp.where(na < nb, na, nb) + xx
    ab = jnp.abs(m) + 1e-30
    o_ref[:] = ab * jax.lax.rsqrt(ab)


def kernel(x, protos):
    batch = x.shape[0]
    return pl.pallas_call(
        _glvq_body,
        out_shape=jax.ShapeDtypeStruct((batch, _NCLS), jnp.float32),
    )(x, protos)


# speedup vs baseline: 1.4267x; 1.0105x over previous
import jax
import jax.numpy as jnp
from jax.experimental import pallas as pl
from jax.experimental.pallas import tpu as pltpu

_NCLS = 512


def _glvq_body(x_ref, p_ref, o_ref):
    x = x_ref[:]
    p = p_ref[:]
    xx = jnp.sum(x * x, axis=1, keepdims=True)
    pp = jnp.sum(p * p, axis=1, keepdims=True)
    ones_x = jnp.ones_like(xx)
    ones_p = jnp.ones_like(pp)
    xa = jnp.concatenate([x * -2.0, xx, ones_x], axis=1)
    pa = jnp.concatenate([p, ones_p, pp], axis=1)
    dn = (((1,), (1,)), ((), ()))
    d2 = jax.lax.dot_general(xa, pa, dn, preferred_element_type=jnp.float32)
    na = d2[:, :_NCLS]
    nb = d2[:, _NCLS:]
    m = jnp.where(na < nb, na, nb)
    ab = jnp.abs(m) + 1e-30
    o_ref[:] = ab * jax.lax.rsqrt(ab)


def kernel(x, protos):
    batch = x.shape[0]
    return pl.pallas_call(
        _glvq_body,
        out_shape=jax.ShapeDtypeStruct((batch, _NCLS), jnp.float32),
    )(x, protos)


# speedup vs baseline: 1.4461x; 1.0136x over previous
import jax
import jax.numpy as jnp
from jax.experimental import pallas as pl
from jax.experimental.pallas import tpu as pltpu

_NCLS = 512
_NB = 2


def _glvq_body(x_ref, p_ref, o_ref):
    x = x_ref[:]
    p = p_ref[:]
    xx = jnp.sum(x * x, axis=1, keepdims=True)
    pp = jnp.sum(p * p, axis=1, keepdims=True)
    ones_x = jnp.ones_like(xx)
    ones_p = jnp.ones_like(pp)
    xa = jnp.concatenate([x * -2.0, xx, ones_x], axis=1)
    pa = jnp.concatenate([p, ones_p, pp], axis=1)
    dn = (((1,), (1,)), ((), ()))
    d2 = jax.lax.dot_general(xa, pa, dn, preferred_element_type=jnp.float32)
    na = d2[:, :_NCLS]
    nb = d2[:, _NCLS:]
    m = jnp.where(na < nb, na, nb)
    ab = jnp.abs(m) + 1e-30
    o_ref[:] = ab * jax.lax.rsqrt(ab)


def kernel(x, protos):
    batch, d = x.shape
    bb = batch // _NB
    return pl.pallas_call(
        _glvq_body,
        grid=(_NB,),
        in_specs=[
            pl.BlockSpec((bb, d), lambda i: (i, 0)),
            pl.BlockSpec(protos.shape, lambda i: (0, 0)),
        ],
        out_specs=pl.BlockSpec((bb, _NCLS), lambda i: (i, 0)),
        out_shape=jax.ShapeDtypeStruct((batch, _NCLS), jnp.float32),
    )(x, protos)
